# baseline (speedup 1.0000x reference)
import functools
import math

import jax
import jax.numpy as jnp
import numpy as np
from jax import lax
from jax.experimental import pallas as pl
from jax.experimental.pallas import tpu as pltpu
from jax.experimental.pallas import tpu_sc as plsc

_VOCAB = 1000000
_D = 64
_B = 1024
_L = 200
_N = _B * _L
_NC = 2
_NS = 16
_NW = _NC * _NS
_CHUNK = 128
_CPW = _N // (_NW * _CHUNK)
_CPL = _B // _CHUNK
_LANES = 16
_P = 5


def _make_pos_enc():
    pe = np.zeros((_L, _D), dtype=np.float32)
    position = np.arange(0.0, _L, dtype=np.float64)[:, None]
    div_term = np.exp(
        np.arange(0.0, _D, 2, dtype=np.float64) * -(math.log(10000.0) / _D))
    pe[:, 0::2] = np.sin(position * div_term).astype(np.float32)
    pe[:, 1::2] = np.cos(position * div_term).astype(np.float32)
    return pe


_PE = _make_pos_enc()

_mesh = plsc.VectorSubcoreMesh(
    core_axis_name="c", subcore_axis_name="s", num_cores=_NC, num_subcores=_NS)


@functools.partial(
    pl.kernel,
    out_type=jax.ShapeDtypeStruct((_B, _L, _D), jnp.float32),
    mesh=_mesh,
    compiler_params=pltpu.CompilerParams(use_tc_tiling_on_sc=False),
    scratch_types=[
        pltpu.VMEM((_CPW, _CHUNK), jnp.int32),
        pltpu.VMEM((_L, _D), jnp.float32),
        pltpu.VMEM((_P, _CHUNK, _D), jnp.float32),
        [pltpu.SemaphoreType.DMA] * _P,
        [pltpu.SemaphoreType.DMA] * _P,
        pltpu.SemaphoreType.DMA,
    ],
)
def _emb_pe_kernel(table_hbm, idx_hbm, pe_hbm, out_hbm,
                   idx_v, pe_v, rows_v, gsems, wsems, isem):
    wid = lax.axis_index("s") * _NC + lax.axis_index("c")
    chunk0 = wid * _CPW

    def idx_fetch(j, carry):
        g = chunk0 + j
        pltpu.async_copy(
            idx_hbm.at[lax.div(g, _CPL), lax.rem(g, _CPL)], idx_v.at[j], isem)
        return carry

    lax.fori_loop(0, _CPW, idx_fetch, 0)
    pltpu.sync_copy(pe_hbm, pe_v)

    def idx_drain(j, carry):
        pltpu.make_async_copy(idx_hbm.at[0, 0], idx_v.at[0], isem).wait()
        return carry

    lax.fori_loop(0, _CPW, idx_drain, 0)

    def gather_start(j, b):
        pltpu.async_copy(table_hbm.at[idx_v.at[j]], rows_v.at[b], gsems[b])

    for b in range(_P):
        gather_start(b, b)

    def outer(s, carry):
        for b in range(_P):
            j = s * _P + b
            g = chunk0 + j
            l = lax.div(g, _CPL)
            b0 = pl.multiple_of(lax.rem(g, _CPL) * _CHUNK, _CHUNK)
            pltpu.make_async_copy(
                table_hbm.at[idx_v.at[j]], rows_v.at[b], gsems[b]).wait()
            pes = [pe_v[l, pl.ds(c * _LANES, _LANES)] for c in range(_D // _LANES)]

            def row_body(r, pes):
                for c in range(_D // _LANES):
                    sl = pl.ds(c * _LANES, _LANES)
                    rows_v[b, r, sl] = rows_v[b, r, sl] * 8.0 + pes[c]
                return pes

            lax.fori_loop(0, _CHUNK, row_body, tuple(pes), unroll=4)
            pltpu.async_copy(
                rows_v.at[b], out_hbm.at[pl.ds(b0, _CHUNK), l], wsems[b])

            @pl.when(s + 1 < _CPW // _P)
            def _():
                pltpu.make_async_copy(
                    rows_v.at[b], out_hbm.at[pl.ds(b0, _CHUNK), l],
                    wsems[b]).wait()
                gather_start(j + _P, b)

        return carry

    lax.fori_loop(0, _CPW // _P, outer, 0)
    for b in range(_P):
        j = _CPW - _P + b
        g = chunk0 + j
        l = lax.div(g, _CPL)
        b0 = pl.multiple_of(lax.rem(g, _CPL) * _CHUNK, _CHUNK)
        pltpu.make_async_copy(
            rows_v.at[b], out_hbm.at[pl.ds(b0, _CHUNK), l], wsems[b]).wait()


def kernel(x, table):
    idx3d = x.T.reshape(_L, _CPL, _CHUNK)
    return _emb_pe_kernel(table, idx3d, _PE)


# speedup vs baseline: 1.0026x; 1.0026x over previous
import functools
import math

import jax
import jax.numpy as jnp
import numpy as np
from jax import lax
from jax.experimental import pallas as pl
from jax.experimental.pallas import tpu as pltpu
from jax.experimental.pallas import tpu_sc as plsc

_VOCAB = 1000000
_D = 64
_B = 1024
_L = 200
_N = _B * _L
_NC = 2
_NS = 16
_NW = _NC * _NS
_CHUNK = 128
_CPW = _N // (_NW * _CHUNK)
_CPL = _B // _CHUNK
_LANES = 16
_P = 5


def _make_pos_enc():
    pe = np.zeros((_L, _D), dtype=np.float32)
    position = np.arange(0.0, _L, dtype=np.float64)[:, None]
    div_term = np.exp(
        np.arange(0.0, _D, 2, dtype=np.float64) * -(math.log(10000.0) / _D))
    pe[:, 0::2] = np.sin(position * div_term).astype(np.float32)
    pe[:, 1::2] = np.cos(position * div_term).astype(np.float32)
    return pe


_PE = _make_pos_enc()

_mesh = plsc.VectorSubcoreMesh(
    core_axis_name="c", subcore_axis_name="s", num_cores=_NC, num_subcores=_NS)


@functools.partial(
    pl.kernel,
    out_type=jax.ShapeDtypeStruct((_B, _L, _D), jnp.float32),
    mesh=_mesh,
    compiler_params=pltpu.CompilerParams(use_tc_tiling_on_sc=False),
    scratch_types=[
        pltpu.VMEM((_CPW, _CHUNK), jnp.int32),
        pltpu.VMEM((_L, _D), jnp.float32),
        pltpu.VMEM((_P, _CHUNK, _D), jnp.float32),
        [pltpu.SemaphoreType.DMA] * _P,
        [pltpu.SemaphoreType.DMA] * _P,
        pltpu.SemaphoreType.DMA,
    ],
)
def _emb_pe_kernel(table_hbm, idx_hbm, pe_hbm, out_hbm,
                   idx_v, pe_v, rows_v, gsems, wsems, isem):
    wid = lax.axis_index("s") * _NC + lax.axis_index("c")
    chunk0 = wid * _CPW

    def idx_fetch(j, carry):
        g = chunk0 + j
        l = lax.div(g, _CPL)
        pltpu.async_copy(
            idx_hbm.at[lax.div(l, 8), lax.rem(g, _CPL), lax.rem(l, 8)],
            idx_v.at[j], isem)
        return carry

    lax.fori_loop(0, _CPW, idx_fetch, 0)
    pltpu.sync_copy(pe_hbm, pe_v)

    def idx_drain(j, carry):
        pltpu.make_async_copy(idx_hbm.at[0, 0, 0], idx_v.at[0], isem).wait()
        return carry

    lax.fori_loop(0, _CPW, idx_drain, 0)

    def gather_start(j, b):
        pltpu.async_copy(table_hbm.at[idx_v.at[j]], rows_v.at[b], gsems[b])

    for b in range(_P):
        gather_start(b, b)

    def outer(s, carry):
        for b in range(_P):
            j = s * _P + b
            g = chunk0 + j
            l = lax.div(g, _CPL)
            b0 = pl.multiple_of(lax.rem(g, _CPL) * _CHUNK, _CHUNK)
            pltpu.make_async_copy(
                table_hbm.at[idx_v.at[j]], rows_v.at[b], gsems[b]).wait()
            pes = [pe_v[l, pl.ds(k * _LANES, _LANES)]
                   for k in range(_D // _LANES)]

            def row_body(r, pes):
                for k in range(_D // _LANES):
                    sl = pl.ds(k * _LANES, _LANES)
                    rows_v[b, r, sl] = rows_v[b, r, sl] * 8.0 + pes[k]
                return pes

            lax.fori_loop(0, _CHUNK, row_body, tuple(pes), unroll=4)
            pltpu.async_copy(
                rows_v.at[b], out_hbm.at[pl.ds(b0, _CHUNK), l], wsems[b])

            @pl.when(s + 1 < _CPW // _P)
            def _():
                pltpu.make_async_copy(
                    rows_v.at[b], out_hbm.at[pl.ds(b0, _CHUNK), l],
                    wsems[b]).wait()
                gather_start(j + _P, b)

        return carry

    lax.fori_loop(0, _CPW // _P, outer, 0)
    for b in range(_P):
        j = _CPW - _P + b
        g = chunk0 + j
        l = lax.div(g, _CPL)
        b0 = pl.multiple_of(lax.rem(g, _CPL) * _CHUNK, _CHUNK)
        pltpu.make_async_copy(
            rows_v.at[b], out_hbm.at[pl.ds(b0, _CHUNK), l], wsems[b]).wait()


def kernel(x, table):
    idx4 = x.reshape(_CPL, _CHUNK, _L // 8, 8).transpose(2, 0, 3, 1)
    return _emb_pe_kernel(table, idx4, _PE)


# speedup vs baseline: 1.0047x; 1.0020x over previous
import functools
import math

import jax
import jax.numpy as jnp
import numpy as np
from jax import lax
from jax.experimental import pallas as pl
from jax.experimental.pallas import tpu as pltpu
from jax.experimental.pallas import tpu_sc as plsc

_VOCAB = 1000000
_D = 64
_B = 1024
_L = 200
_N = _B * _L
_NC = 2
_NS = 16
_NW = _NC * _NS
_CHUNK = 128
_CPW = _N // (_NW * _CHUNK)
_CPL = _B // _CHUNK
_LANES = 16
_P = 5


def _make_pos_enc():
    pe = np.zeros((_L, _D), dtype=np.float32)
    position = np.arange(0.0, _L, dtype=np.float64)[:, None]
    div_term = np.exp(
        np.arange(0.0, _D, 2, dtype=np.float64) * -(math.log(10000.0) / _D))
    pe[:, 0::2] = np.sin(position * div_term).astype(np.float32)
    pe[:, 1::2] = np.cos(position * div_term).astype(np.float32)
    return pe


_PE = _make_pos_enc()

_mesh = plsc.VectorSubcoreMesh(
    core_axis_name="c", subcore_axis_name="s", num_cores=_NC, num_subcores=_NS)

_NT = (_L // 8) * _CPL
_TPW = (_NT + _NW - 1) // _NW


@functools.partial(
    pl.kernel,
    out_type=jax.ShapeDtypeStruct((_L // 8, _CPL, 8, _CHUNK), jnp.int32),
    mesh=_mesh,
    compiler_params=pltpu.CompilerParams(use_tc_tiling_on_sc=True),
    scratch_types=[
        pltpu.VMEM((_TPW, 8, _CHUNK), jnp.int32),
        pltpu.SemaphoreType.DMA,
        pltpu.SemaphoreType.DMA,
    ],
)
def _retile_kernel(xt_hbm, out_hbm, tile_v, isem, osem):
    wid = lax.axis_index("s") * _NC + lax.axis_index("c")
    for i in range(_TPW):
        t = wid + _NW * i

        @pl.when(t < _NT)
        def _():
            pltpu.async_copy(
                xt_hbm.at[pl.ds(lax.div(t, _CPL) * 8, 8),
                          pl.ds(lax.rem(t, _CPL) * _CHUNK, _CHUNK)],
                tile_v.at[i], isem)

    for i in range(_TPW):
        t = wid + _NW * i

        @pl.when(t < _NT)
        def _():
            pltpu.make_async_copy(
                xt_hbm.at[pl.ds(0, 8), pl.ds(0, _CHUNK)],
                tile_v.at[i], isem).wait()
            pltpu.async_copy(
                tile_v.at[i],
                out_hbm.at[lax.div(t, _CPL), lax.rem(t, _CPL)], osem)

    for i in range(_TPW):
        t = wid + _NW * i

        @pl.when(t < _NT)
        def _():
            pltpu.make_async_copy(
                tile_v.at[i],
                out_hbm.at[lax.div(t, _CPL), lax.rem(t, _CPL)], osem).wait()


@functools.partial(
    pl.kernel,
    out_type=jax.ShapeDtypeStruct((_B, _L, _D), jnp.float32),
    mesh=_mesh,
    compiler_params=pltpu.CompilerParams(use_tc_tiling_on_sc=False),
    scratch_types=[
        pltpu.VMEM((_CPW, _CHUNK), jnp.int32),
        pltpu.VMEM((_L, _D), jnp.float32),
        pltpu.VMEM((_P, _CHUNK, _D), jnp.float32),
        [pltpu.SemaphoreType.DMA] * _P,
        [pltpu.SemaphoreType.DMA] * _P,
        pltpu.SemaphoreType.DMA,
    ],
)
def _emb_pe_kernel(table_hbm, idx_hbm, pe_hbm, out_hbm,
                   idx_v, pe_v, rows_v, gsems, wsems, isem):
    wid = lax.axis_index("s") * _NC + lax.axis_index("c")
    chunk0 = wid * _CPW

    def idx_fetch(j, carry):
        g = chunk0 + j
        l = lax.div(g, _CPL)
        pltpu.async_copy(
            idx_hbm.at[lax.div(l, 8), lax.rem(g, _CPL), lax.rem(l, 8)],
            idx_v.at[j], isem)
        return carry

    lax.fori_loop(0, _CPW, idx_fetch, 0)
    pltpu.sync_copy(pe_hbm, pe_v)

    def idx_drain(j, carry):
        pltpu.make_async_copy(idx_hbm.at[0, 0, 0], idx_v.at[0], isem).wait()
        return carry

    lax.fori_loop(0, _CPW, idx_drain, 0)

    def gather_start(j, b):
        pltpu.async_copy(table_hbm.at[idx_v.at[j]], rows_v.at[b], gsems[b])

    for b in range(_P):
        gather_start(b, b)

    def outer(s, carry):
        for b in range(_P):
            j = s * _P + b
            g = chunk0 + j
            l = lax.div(g, _CPL)
            b0 = pl.multiple_of(lax.rem(g, _CPL) * _CHUNK, _CHUNK)
            pltpu.make_async_copy(
                table_hbm.at[idx_v.at[j]], rows_v.at[b], gsems[b]).wait()
            pes = [pe_v[l, pl.ds(k * _LANES, _LANES)]
                   for k in range(_D // _LANES)]

            def row_body(r, pes):
                for k in range(_D // _LANES):
                    sl = pl.ds(k * _LANES, _LANES)
                    rows_v[b, r, sl] = rows_v[b, r, sl] * 8.0 + pes[k]
                return pes

            lax.fori_loop(0, _CHUNK, row_body, tuple(pes), unroll=4)
            pltpu.async_copy(
                rows_v.at[b], out_hbm.at[pl.ds(b0, _CHUNK), l], wsems[b])

            @pl.when(s + 1 < _CPW // _P)
            def _():
                pltpu.make_async_copy(
                    rows_v.at[b], out_hbm.at[pl.ds(b0, _CHUNK), l],
                    wsems[b]).wait()
                gather_start(j + _P, b)

        return carry

    lax.fori_loop(0, _CPW // _P, outer, 0)
    for b in range(_P):
        j = _CPW - _P + b
        g = chunk0 + j
        l = lax.div(g, _CPL)
        b0 = pl.multiple_of(lax.rem(g, _CPL) * _CHUNK, _CHUNK)
        pltpu.make_async_copy(
            rows_v.at[b], out_hbm.at[pl.ds(b0, _CHUNK), l], wsems[b]).wait()


def kernel(x, table):
    idx4 = _retile_kernel(x.T)
    return _emb_pe_kernel(table, idx4, _PE)
